# baseline (speedup 1.0000x reference)
import functools

import jax
import jax.numpy as jnp
from jax import lax
from jax.experimental import pallas as pl
from jax.experimental.pallas import tpu as pltpu
from jax.experimental.pallas import tpu_sc as plsc

_D = 64
_V = 33
_NC = 2
_NS = 16
_NW = _NC * _NS
_CHUNK = 512


def _make_sc_call(S: int, P: int):
    s_per_w = S // _NW
    n_chunks = P // _CHUNK
    mesh = plsc.VectorSubcoreMesh(core_axis_name="c", subcore_axis_name="s")

    @functools.partial(
        pl.kernel,
        out_type=jax.ShapeDtypeStruct((S, P, _D), jnp.float32),
        mesh=mesh,
        scratch_types=[
            pltpu.VMEM((_V, _D), jnp.float32),
            pltpu.VMEM((_CHUNK,), jnp.int32),
            pltpu.VMEM((_CHUNK, _D), jnp.float32),
        ],
    )
    def sc_kernel(logits_hbm, seq_hbm, mask_hbm, out_hbm, mask_v, idx_v, buf_v):
        wid = lax.axis_index("s") * _NC + lax.axis_index("c")
        pltpu.sync_copy(mask_hbm, mask_v)

        def chunk_body(k, carry):
            s = wid * s_per_w + k // n_chunks
            p0 = (k % n_chunks) * _CHUNK
            pltpu.sync_copy(logits_hbm.at[s, pl.ds(p0, _CHUNK)], buf_v)
            pltpu.sync_copy(seq_hbm.at[s, pl.ds(p0, _CHUNK)], idx_v)

            def row_body(g, c2):
                t16 = idx_v[pl.ds(g * 16, 16)]
                for j in range(16):
                    t = t16[j]
                    for q in range(_D // 16):
                        m = mask_v[t, pl.ds(q * 16, 16)]
                        plsc.addupdate(buf_v.at[g * 16 + j, pl.ds(q * 16, 16)], m)
                return c2

            lax.fori_loop(0, _CHUNK // 16, row_body, 0, unroll=False)
            pltpu.sync_copy(buf_v, out_hbm.at[s, pl.ds(p0, _CHUNK)])
            return carry

        lax.fori_loop(0, s_per_w * n_chunks, chunk_body, 0, unroll=False)

    return sc_kernel


@jax.jit
def kernel(logits_SPT, seq_SP, valid_output_mask_TiTo):
    S, P, T = logits_SPT.shape
    seq = seq_SP.astype(jnp.int32)
    mask = valid_output_mask_TiTo.astype(jnp.float32)
    return _make_sc_call(S, P)(logits_SPT, seq, mask)
